# Initial kernel scaffold; baseline (speedup 1.0000x reference)
import jax
import jax.numpy as jnp
from jax.experimental import pallas as pl


def kernel(denseFeat, catFeat, emb_table, W, b):
    raise NotImplementedError("write your pallas kernel here")


# speedup vs baseline: 44.0304x; 44.0304x over previous
import functools

import jax
import jax.numpy as jnp
from jax import lax
from jax.experimental import pallas as pl
from jax.experimental.pallas import tpu as pltpu
from jax.experimental.pallas import tpu_sc as plsc

NC = 2
NS = 16
NW = NC * NS
LANES = 16

B = 16384
L = 200
N = B * L
PER_W = N // NW
CHUNK = 10240
N_CHUNKS = PER_W // CHUNK
GROUPS = CHUNK // LANES


def _body(dense_hbm, cat_hbm, emb_hbm, wb_hbm, out_hbm,
          emb_v, wb_v, lut_v, dbuf, cbuf, obuf):
    wid = lax.axis_index("s") * NC + lax.axis_index("c")
    base = wid * PER_W

    pltpu.sync_copy(emb_hbm, emb_v)
    pltpu.sync_copy(wb_hbm, wb_v)
    iota = lax.iota(jnp.int32, LANES)
    e0 = plsc.load_gather(emb_v, [jnp.minimum(iota * 2, 14)])
    e1 = plsc.load_gather(emb_v, [jnp.minimum(iota * 2 + 1, 15)])
    w0v = wb_v[pl.ds(0, LANES)]
    w1v = wb_v[pl.ds(LANES, LANES)]
    w2v = wb_v[pl.ds(2 * LANES, LANES)]
    bv = wb_v[pl.ds(3 * LANES, LANES)]
    lut_v[...] = e0 * w1v + e1 * w2v + bv

    def chunk_body(ci, _):
        off = base + ci * CHUNK
        pltpu.sync_copy(dense_hbm.at[pl.ds(off, CHUNK)], dbuf)
        pltpu.sync_copy(cat_hbm.at[pl.ds(off, CHUNK)], cbuf)

        def vec_body(i, _):
            s = pl.ds(i * LANES, LANES)
            d = dbuf[s]
            c = cbuf[s]
            t = plsc.load_gather(lut_v, [c])
            x = d * w0v + t
            obuf[s] = 1.0 / (1.0 + jnp.exp(-x))
            return 0

        lax.fori_loop(0, GROUPS, vec_body, 0, unroll=4)
        pltpu.sync_copy(obuf, out_hbm.at[pl.ds(off, CHUNK)])
        return 0

    lax.fori_loop(0, N_CHUNKS, chunk_body, 0)


_mesh = plsc.VectorSubcoreMesh(
    core_axis_name="c", subcore_axis_name="s", num_cores=NC, num_subcores=NS)

_sc_call = functools.partial(
    pl.kernel,
    out_type=jax.ShapeDtypeStruct((N,), jnp.float32),
    mesh=_mesh,
    compiler_params=pltpu.CompilerParams(needs_layout_passes=False),
    scratch_types=[
        pltpu.VMEM((LANES,), jnp.float32),
        pltpu.VMEM((4 * LANES,), jnp.float32),
        pltpu.VMEM((LANES,), jnp.float32),
        pltpu.VMEM((CHUNK,), jnp.float32),
        pltpu.VMEM((CHUNK,), jnp.int32),
        pltpu.VMEM((CHUNK,), jnp.float32),
    ],
)(_body)


def kernel(denseFeat, catFeat, emb_table, W, b):
    dense = denseFeat.reshape(-1)
    cat = catFeat.astype(jnp.int32).reshape(-1)
    emb16 = jnp.zeros((LANES,), jnp.float32).at[:10].set(emb_table.reshape(-1))
    wb64 = jnp.concatenate([
        jnp.broadcast_to(W[0, 0], (LANES,)),
        jnp.broadcast_to(W[1, 0], (LANES,)),
        jnp.broadcast_to(W[2, 0], (LANES,)),
        jnp.broadcast_to(b[0], (LANES,)),
    ]).astype(jnp.float32)
    out_flat = _sc_call(dense, cat, emb16, wb64)
    return out_flat.reshape(denseFeat.shape[0], denseFeat.shape[1], 1)


# speedup vs baseline: 47.2405x; 1.0729x over previous
import functools

import jax
import jax.numpy as jnp
from jax import lax
from jax.experimental import pallas as pl
from jax.experimental.pallas import tpu as pltpu
from jax.experimental.pallas import tpu_sc as plsc

NC = 2
NS = 16
NW = NC * NS
LANES = 16

B = 16384
L = 200
N = B * L
PER_W = N // NW
CHUNK = 10240
N_CHUNKS = PER_W // CHUNK
GROUPS = CHUNK // LANES


def _body(dense_hbm, cat_hbm, emb_hbm, wb_hbm, out_hbm,
          emb_v, wb_v, lut_v, d0, d1, c0, c1, o0, o1,
          sd0, sd1, sc0, sc1, so0, so1):
    wid = lax.axis_index("s") * NC + lax.axis_index("c")
    base = wid * PER_W
    dbufs, cbufs, obufs = [d0, d1], [c0, c1], [o0, o1]
    dsems, csems, osems = [sd0, sd1], [sc0, sc1], [so0, so1]

    pltpu.sync_copy(emb_hbm, emb_v)
    pltpu.sync_copy(wb_hbm, wb_v)
    iota = lax.iota(jnp.int32, LANES)
    e0 = plsc.load_gather(emb_v, [jnp.minimum(iota * 2, 14)])
    e1 = plsc.load_gather(emb_v, [jnp.minimum(iota * 2 + 1, 15)])
    w0v = wb_v[pl.ds(0, LANES)]
    w1v = wb_v[pl.ds(LANES, LANES)]
    w2v = wb_v[pl.ds(2 * LANES, LANES)]
    bv = wb_v[pl.ds(3 * LANES, LANES)]
    lut_v[...] = e0 * w1v + e1 * w2v + bv

    def start_in(ci):
        k = ci % 2
        off = base + ci * CHUNK
        return (
            pltpu.async_copy(dense_hbm.at[pl.ds(off, CHUNK)], dbufs[k], dsems[k]),
            pltpu.async_copy(cat_hbm.at[pl.ds(off, CHUNK)], cbufs[k], csems[k]),
        )

    in_descs = {0: start_in(0)}
    out_descs = {}
    for ci in range(N_CHUNKS):
        k = ci % 2
        if ci + 1 < N_CHUNKS:
            in_descs[ci + 1] = start_in(ci + 1)
        din, cin = in_descs.pop(ci)
        din.wait()
        cin.wait()
        if ci >= 2:
            out_descs.pop(ci - 2).wait()
        dbuf, cbuf, obuf = dbufs[k], cbufs[k], obufs[k]

        def vec_body(i, _, dbuf=dbuf, cbuf=cbuf, obuf=obuf):
            s = pl.ds(i * LANES, LANES)
            d = dbuf[s]
            c = cbuf[s]
            t = plsc.load_gather(lut_v, [c])
            x = d * w0v + t
            obuf[s] = 1.0 / (1.0 + jnp.exp(-x))
            return 0

        lax.fori_loop(0, GROUPS, vec_body, 0, unroll=4)
        out_descs[ci] = pltpu.async_copy(
            obuf, out_hbm.at[pl.ds(base + ci * CHUNK, CHUNK)], osems[k])
    out_descs.pop(N_CHUNKS - 2).wait()
    out_descs.pop(N_CHUNKS - 1).wait()


_mesh = plsc.VectorSubcoreMesh(
    core_axis_name="c", subcore_axis_name="s", num_cores=NC, num_subcores=NS)

_sc_call = functools.partial(
    pl.kernel,
    out_type=jax.ShapeDtypeStruct((N,), jnp.float32),
    mesh=_mesh,
    compiler_params=pltpu.CompilerParams(needs_layout_passes=False),
    scratch_types=[
        pltpu.VMEM((LANES,), jnp.float32),
        pltpu.VMEM((4 * LANES,), jnp.float32),
        pltpu.VMEM((LANES,), jnp.float32),
        pltpu.VMEM((CHUNK,), jnp.float32),
        pltpu.VMEM((CHUNK,), jnp.float32),
        pltpu.VMEM((CHUNK,), jnp.int32),
        pltpu.VMEM((CHUNK,), jnp.int32),
        pltpu.VMEM((CHUNK,), jnp.float32),
        pltpu.VMEM((CHUNK,), jnp.float32),
        pltpu.SemaphoreType.DMA,
        pltpu.SemaphoreType.DMA,
        pltpu.SemaphoreType.DMA,
        pltpu.SemaphoreType.DMA,
        pltpu.SemaphoreType.DMA,
        pltpu.SemaphoreType.DMA,
    ],
)(_body)


def kernel(denseFeat, catFeat, emb_table, W, b):
    dense = denseFeat.reshape(-1)
    cat = catFeat.astype(jnp.int32).reshape(-1)
    emb16 = jnp.zeros((LANES,), jnp.float32).at[:10].set(emb_table.reshape(-1))
    wb64 = jnp.concatenate([
        jnp.broadcast_to(W[0, 0], (LANES,)),
        jnp.broadcast_to(W[1, 0], (LANES,)),
        jnp.broadcast_to(W[2, 0], (LANES,)),
        jnp.broadcast_to(b[0], (LANES,)),
    ]).astype(jnp.float32)
    out_flat = _sc_call(dense, cat, emb16, wb64)
    return out_flat.reshape(denseFeat.shape[0], denseFeat.shape[1], 1)


# speedup vs baseline: 92.2589x; 1.9530x over previous
import functools

import jax
import jax.numpy as jnp
from jax import lax
from jax.experimental import pallas as pl
from jax.experimental.pallas import tpu as pltpu
from jax.experimental.pallas import tpu_sc as plsc

NC = 2
NS = 16
NW = NC * NS
LANES = 16

B = 16384
L = 200
N = B * L
PER_W = N // NW
CHUNK = 10240
N_CHUNKS = PER_W // CHUNK
GROUPS = CHUNK // LANES


def _body(dense_hbm, cat_hbm, emb_hbm, wb_hbm, out_hbm,
          emb_v, wb_v, lut_v, d0, d1, c0, c1, o0, o1,
          sd0, sd1, sc0, sc1, so0, so1):
    wid = lax.axis_index("s") * NC + lax.axis_index("c")
    base = wid * PER_W
    dbufs, cbufs, obufs = [d0, d1], [c0, c1], [o0, o1]
    dsems, csems, osems = [sd0, sd1], [sc0, sc1], [so0, so1]

    pltpu.sync_copy(emb_hbm, emb_v)
    pltpu.sync_copy(wb_hbm, wb_v)
    iota = lax.iota(jnp.int32, LANES)
    e0 = plsc.load_gather(emb_v, [jnp.minimum(iota * 2, 14)])
    e1 = plsc.load_gather(emb_v, [jnp.minimum(iota * 2 + 1, 15)])
    w0v = wb_v[pl.ds(0, LANES)]
    w1v = wb_v[pl.ds(LANES, LANES)]
    w2v = wb_v[pl.ds(2 * LANES, LANES)]
    bv = wb_v[pl.ds(3 * LANES, LANES)]
    lut_v[...] = e0 * w1v + e1 * w2v + bv

    def start_in(ci):
        k = ci % 2
        off = base + ci * CHUNK
        return (
            pltpu.async_copy(dense_hbm.at[pl.ds(off, CHUNK)], dbufs[k], dsems[k]),
            pltpu.async_copy(cat_hbm.at[pl.ds(off, CHUNK)], cbufs[k], csems[k]),
        )

    in_descs = {0: start_in(0)}
    out_descs = {}
    for ci in range(N_CHUNKS):
        k = ci % 2
        if ci + 1 < N_CHUNKS:
            in_descs[ci + 1] = start_in(ci + 1)
        din, cin = in_descs.pop(ci)
        din.wait()
        cin.wait()
        if ci >= 2:
            out_descs.pop(ci - 2).wait()
        dbuf, cbuf, obuf = dbufs[k], cbufs[k], obufs[k]

        @plsc.parallel_loop(0, CHUNK, step=LANES, unroll=8)
        def _(i, dbuf=dbuf, cbuf=cbuf, obuf=obuf):
            s = pl.ds(i, LANES)
            d = dbuf[s]
            c = cbuf[s]
            t = plsc.load_gather(lut_v, [c])
            x = d * w0v + t
            obuf[s] = 1.0 / (1.0 + jnp.exp(-x))
        out_descs[ci] = pltpu.async_copy(
            obuf, out_hbm.at[pl.ds(base + ci * CHUNK, CHUNK)], osems[k])
    out_descs.pop(N_CHUNKS - 2).wait()
    out_descs.pop(N_CHUNKS - 1).wait()


_mesh = plsc.VectorSubcoreMesh(
    core_axis_name="c", subcore_axis_name="s", num_cores=NC, num_subcores=NS)

_sc_call = functools.partial(
    pl.kernel,
    out_type=jax.ShapeDtypeStruct((N,), jnp.float32),
    mesh=_mesh,
    compiler_params=pltpu.CompilerParams(needs_layout_passes=False),
    scratch_types=[
        pltpu.VMEM((LANES,), jnp.float32),
        pltpu.VMEM((4 * LANES,), jnp.float32),
        pltpu.VMEM((LANES,), jnp.float32),
        pltpu.VMEM((CHUNK,), jnp.float32),
        pltpu.VMEM((CHUNK,), jnp.float32),
        pltpu.VMEM((CHUNK,), jnp.int32),
        pltpu.VMEM((CHUNK,), jnp.int32),
        pltpu.VMEM((CHUNK,), jnp.float32),
        pltpu.VMEM((CHUNK,), jnp.float32),
        pltpu.SemaphoreType.DMA,
        pltpu.SemaphoreType.DMA,
        pltpu.SemaphoreType.DMA,
        pltpu.SemaphoreType.DMA,
        pltpu.SemaphoreType.DMA,
        pltpu.SemaphoreType.DMA,
    ],
)(_body)


def kernel(denseFeat, catFeat, emb_table, W, b):
    dense = denseFeat.reshape(-1)
    cat = catFeat.astype(jnp.int32).reshape(-1)
    emb16 = jnp.zeros((LANES,), jnp.float32).at[:10].set(emb_table.reshape(-1))
    wb64 = jnp.concatenate([
        jnp.broadcast_to(W[0, 0], (LANES,)),
        jnp.broadcast_to(W[1, 0], (LANES,)),
        jnp.broadcast_to(W[2, 0], (LANES,)),
        jnp.broadcast_to(b[0], (LANES,)),
    ]).astype(jnp.float32)
    out_flat = _sc_call(dense, cat, emb16, wb64)
    return out_flat.reshape(denseFeat.shape[0], denseFeat.shape[1], 1)


# speedup vs baseline: 123.3252x; 1.3367x over previous
import functools

import jax
import jax.numpy as jnp
from jax import lax
from jax.experimental import pallas as pl
from jax.experimental.pallas import tpu as pltpu
from jax.experimental.pallas import tpu_sc as plsc

NC = 2
NS = 16
NW = NC * NS
LANES = 16

B = 16384
L = 200
ROWS_W = B // NW
RB = 64
N_BLOCKS = ROWS_W // RB
NGROUPS = 13


def _body(dense_hbm, cat_hbm, emb_hbm, wb_hbm, out_hbm,
          emb_v, wb_v, lut_v, d0, d1, c0, c1, o0, o1,
          sd0, sd1, sc0, sc1, so0, so1):
    wid = lax.axis_index("s") * NC + lax.axis_index("c")
    r0 = wid * ROWS_W
    dbufs, cbufs, obufs = [d0, d1], [c0, c1], [o0, o1]
    dsems, csems, osems = [sd0, sd1], [sc0, sc1], [so0, so1]

    pltpu.sync_copy(emb_hbm, emb_v)
    pltpu.sync_copy(wb_hbm, wb_v)
    iota = lax.iota(jnp.int32, LANES)
    e0 = plsc.load_gather(emb_v, [jnp.minimum(iota * 2, 14)])
    e1 = plsc.load_gather(emb_v, [jnp.minimum(iota * 2 + 1, 15)])
    nw0v = wb_v[pl.ds(0, LANES)]
    w1v = wb_v[pl.ds(LANES, LANES)]
    w2v = wb_v[pl.ds(2 * LANES, LANES)]
    bv = wb_v[pl.ds(3 * LANES, LANES)]
    lut_v[...] = -(e0 * w1v + e1 * w2v + bv)

    def start_in(blk, k):
        r = r0 + blk * RB
        pltpu.async_copy(dense_hbm.at[pl.ds(r, RB)], dbufs[k], dsems[k])
        pltpu.async_copy(cat_hbm.at[pl.ds(r, RB)], cbufs[k], csems[k])

    def wait_in(k):
        pltpu.make_async_copy(dense_hbm.at[pl.ds(r0, RB)], dbufs[k], dsems[k]).wait()
        pltpu.make_async_copy(cat_hbm.at[pl.ds(r0, RB)], cbufs[k], csems[k]).wait()

    def start_out(blk, k):
        pltpu.async_copy(obufs[k], out_hbm.at[pl.ds(r0 + blk * RB, RB)], osems[k])

    def wait_out(k):
        pltpu.make_async_copy(obufs[k], out_hbm.at[pl.ds(r0, RB)], osems[k]).wait()

    def compute(k):
        dbuf, cbuf, obuf = dbufs[k], cbufs[k], obufs[k]

        @plsc.parallel_loop(0, RB, step=1, unroll=2)
        def _(i):
            for g in range(NGROUPS):
                s = pl.ds(g * LANES if g < NGROUPS - 1 else L - LANES, LANES)
                d = dbuf[i, s]
                c = cbuf[i, s]
                nt = plsc.load_gather(lut_v, [c])
                obuf[i, s] = 1.0 / (1.0 + jnp.exp(d * nw0v + nt))

    n_pairs = N_BLOCKS // 2
    start_in(0, 0)

    def pair_body(p, _):
        blk0 = 2 * p
        start_in(blk0 + 1, 1)
        wait_in(0)

        @pl.when(p > 0)
        def _():
            wait_out(0)

        compute(0)
        start_out(blk0, 0)

        @pl.when(p + 1 < n_pairs)
        def _():
            start_in(blk0 + 2, 0)

        wait_in(1)

        @pl.when(p > 0)
        def _():
            wait_out(1)

        compute(1)
        start_out(blk0 + 1, 1)
        return 0

    lax.fori_loop(0, n_pairs, pair_body, 0)
    wait_out(0)
    wait_out(1)


_mesh = plsc.VectorSubcoreMesh(
    core_axis_name="c", subcore_axis_name="s", num_cores=NC, num_subcores=NS)

_sc_call = functools.partial(
    pl.kernel,
    out_type=jax.ShapeDtypeStruct((B, L), jnp.float32),
    mesh=_mesh,
    compiler_params=pltpu.CompilerParams(
        needs_layout_passes=False, use_tc_tiling_on_sc=True),
    scratch_types=[
        pltpu.VMEM((LANES,), jnp.float32),
        pltpu.VMEM((4 * LANES,), jnp.float32),
        pltpu.VMEM((LANES,), jnp.float32),
        pltpu.VMEM((RB, L), jnp.float32),
        pltpu.VMEM((RB, L), jnp.float32),
        pltpu.VMEM((RB, L), jnp.int32),
        pltpu.VMEM((RB, L), jnp.int32),
        pltpu.VMEM((RB, L), jnp.float32),
        pltpu.VMEM((RB, L), jnp.float32),
        pltpu.SemaphoreType.DMA,
        pltpu.SemaphoreType.DMA,
        pltpu.SemaphoreType.DMA,
        pltpu.SemaphoreType.DMA,
        pltpu.SemaphoreType.DMA,
        pltpu.SemaphoreType.DMA,
    ],
)(_body)


def kernel(denseFeat, catFeat, emb_table, W, b):
    cat = catFeat.astype(jnp.int32)
    emb16 = jnp.zeros((LANES,), jnp.float32).at[:10].set(emb_table.reshape(-1))
    wb64 = jnp.concatenate([
        jnp.broadcast_to(-W[0, 0], (LANES,)),
        jnp.broadcast_to(W[1, 0], (LANES,)),
        jnp.broadcast_to(W[2, 0], (LANES,)),
        jnp.broadcast_to(b[0], (LANES,)),
    ]).astype(jnp.float32)
    out = _sc_call(denseFeat, cat, emb16, wb64)
    return out[..., None]
